# baseline (speedup 1.0000x reference)
import jax
import jax.numpy as jnp
from jax import lax
from jax.experimental import pallas as pl
from jax.experimental.pallas import tpu as pltpu
from jax.experimental.pallas import tpu_sc as plsc

_V = 100000
_NF = 26
_D = 32
_EPS = 1e-5
_L = 16
_FEAT = _NF * _D
_VP = 100016
_GROWS = _NF * _VP // 4
_RPB = 16672
_RPG = _RPB // 4
_NBLK = _VP // _RPB


def _repack(tables):
    def body(t_ref, o_ref):
        t3 = t_ref[0].reshape(_RPG, 4, _D)
        for k in range(4):
            o_ref[:, k * _D:(k + 1) * _D] = t3[:, k, :]

    return pl.pallas_call(
        body,
        grid=(_NF, _NBLK),
        in_specs=[pl.BlockSpec((1, _RPB, _D), lambda f, c: (f, c, 0))],
        out_specs=pl.BlockSpec((_RPG, 128), lambda f, c: (f * _NBLK + c, 0)),
        out_shape=jax.ShapeDtypeStruct((_GROWS, 128), jnp.float32),
    )(tables)


def _layernorm(og, gamma, beta, B):
    RB = 128
    XR = RB * _FEAT // 128

    def body(g_ref, b_ref, x_ref, o_ref):
        x3 = x_ref[...].reshape(RB // 2, 13, 128)

        def rows(c, lo, hi):
            return x3[:, c, lo:hi]

        h_even = jnp.concatenate(
            [rows(c, 0, 128) for c in range(6)] + [rows(6, 0, 64)], axis=1)
        h_odd = jnp.concatenate(
            [rows(6, 64, 128)] + [rows(7 + c, 0, 128) for c in range(6)],
            axis=1)
        h = jnp.stack([h_even, h_odd], axis=1).reshape(RB, _FEAT)
        mu = jnp.mean(h, axis=1, keepdims=True)
        var = jnp.mean(h * h, axis=1, keepdims=True) - mu * mu
        hn = (h - mu) * lax.rsqrt(var + _EPS)
        o_ref[...] = hn * g_ref[...][None, :] + b_ref[...][None, :]

    return pl.pallas_call(
        body,
        grid=(B // RB,),
        in_specs=[
            pl.BlockSpec((_FEAT,), lambda r: (0,)),
            pl.BlockSpec((_FEAT,), lambda r: (0,)),
            pl.BlockSpec((XR, 128), lambda r: (r, 0)),
        ],
        out_specs=pl.BlockSpec((RB, _FEAT), lambda r: (r, 0)),
        out_shape=jax.ShapeDtypeStruct((B, _FEAT), jnp.float32),
    )(gamma, beta, og)


def kernel(x, tables, gamma, beta):
    B = x.shape[0]
    info = plsc.get_sparse_core_info()
    NC, NS = info.num_cores, info.num_subcores
    NW = NC * NS
    LPW = B * _NF // NW
    CL = 128
    NCH = LPW // CL
    ORC = CL * _D // 128
    IR = B * _NF // 128
    IRW = IR // NW

    offs = (jnp.arange(_NF, dtype=jnp.int32) * _VP)[None, :]
    idxp = (jnp.clip(x, 0, _V) + offs).reshape(IR, 128)
    tabp = jnp.pad(tables, ((0, 0), (0, _VP - _V - 1), (0, 32))).reshape(-1, 64)

    mesh = plsc.VectorSubcoreMesh(core_axis_name="c", subcore_axis_name="s")

    @pl.kernel(
        mesh=mesh,
        compiler_params=pltpu.CompilerParams(use_tc_tiling_on_sc=False),
        out_type=jax.ShapeDtypeStruct((B * _FEAT // 128, 128), jnp.float32),
        scratch_types=[
            pltpu.VMEM((IRW, 128), jnp.int32),
            pltpu.VMEM((CL,), jnp.int32),
            pltpu.VMEM((CL, 64), jnp.float32),
            pltpu.VMEM((ORC, 128), jnp.float32),
            pltpu.SemaphoreType.DMA,
        ],
    )
    def gather(idx_hbm, tab_hbm, out_hbm, idx_v, gsc_v, emb_v, outb_v, gsem):
        wid = lax.axis_index("s") * NC + lax.axis_index("c")
        pltpu.sync_copy(idx_hbm.at[pl.ds(wid * IRW, IRW)], idx_v)

        def chunk_body(c, carry):
            for k in range(CL // _L):
                gsc_v[pl.ds(k * _L, _L)] = idx_v[c, pl.ds(k * _L, _L)]
            pltpu.async_copy(tab_hbm.at[gsc_v], emb_v, gsem).wait()
            for lk in range(CL):
                for h in range(2):
                    outb_v[lk >> 2, pl.ds((lk & 3) * _D + h * _L, _L)] = (
                        emb_v[lk, pl.ds(h * _L, _L)])
            pltpu.sync_copy(outb_v,
                            out_hbm.at[pl.ds(wid * (LPW * _D // 128) + c * ORC,
                                             ORC)])
            return carry

        lax.fori_loop(0, NCH, chunk_body, 0)

    og = gather(idxp, tabp)
    return _layernorm(og, gamma, beta, B)


# speedup vs baseline: 1.6300x; 1.6300x over previous
import jax
import jax.numpy as jnp
from jax import lax
from jax.experimental import pallas as pl
from jax.experimental.pallas import tpu as pltpu
from jax.experimental.pallas import tpu_sc as plsc

_V = 100000
_NF = 26
_D = 32
_EPS = 1e-5
_L = 16
_FEAT = _NF * _D
_VP = 100008
_GROWS = _NF * _VP // 4
_RPB = 16672
_RPG = _RPB // 4
_NBLK = _VP // _RPB


def _repack(tables):
    def body(t_ref, o_ref):
        t3 = t_ref[0].reshape(_RPG, 4, _D)
        for k in range(4):
            o_ref[:, k * _D:(k + 1) * _D] = t3[:, k, :]

    return pl.pallas_call(
        body,
        grid=(_NF, _NBLK),
        in_specs=[pl.BlockSpec((1, _RPB, _D), lambda f, c: (f, c, 0))],
        out_specs=pl.BlockSpec((_RPG, 128), lambda f, c: (f * _NBLK + c, 0)),
        out_shape=jax.ShapeDtypeStruct((_GROWS, 128), jnp.float32),
    )(tables)


def _layernorm(og, gamma, beta, B):
    RB = 128
    XR = RB * _FEAT // 128

    def body(g_ref, b_ref, x_ref, o_ref):
        x3 = x_ref[...].reshape(RB // 2, 13, 128)

        def rows(c, lo, hi):
            return x3[:, c, lo:hi]

        h_even = jnp.concatenate(
            [rows(c, 0, 128) for c in range(6)] + [rows(6, 0, 64)], axis=1)
        h_odd = jnp.concatenate(
            [rows(6, 64, 128)] + [rows(7 + c, 0, 128) for c in range(6)],
            axis=1)
        h = jnp.stack([h_even, h_odd], axis=1).reshape(RB, _FEAT)
        mu = jnp.mean(h, axis=1, keepdims=True)
        var = jnp.mean(h * h, axis=1, keepdims=True) - mu * mu
        hn = (h - mu) * lax.rsqrt(var + _EPS)
        o_ref[...] = hn * g_ref[...][None, :] + b_ref[...][None, :]

    return pl.pallas_call(
        body,
        grid=(B // RB,),
        in_specs=[
            pl.BlockSpec((_FEAT,), lambda r: (0,)),
            pl.BlockSpec((_FEAT,), lambda r: (0,)),
            pl.BlockSpec((XR, 128), lambda r: (r, 0)),
        ],
        out_specs=pl.BlockSpec((RB, _FEAT), lambda r: (r, 0)),
        out_shape=jax.ShapeDtypeStruct((B, _FEAT), jnp.float32),
    )(gamma, beta, og)


def kernel(x, tables, gamma, beta):
    B = x.shape[0]
    info = plsc.get_sparse_core_info()
    NC, NS = info.num_cores, info.num_subcores
    NW = NC * NS
    LPW = B * _NF // NW
    CL = 128
    NCH = LPW // CL
    ORC = CL * _D // 128
    IR = B * _NF // 128
    IRW = IR // NW

    offs = (jnp.arange(_NF, dtype=jnp.int32) * _VP)[None, :]
    idxp = (jnp.clip(x, 0, _V) + offs).reshape(IR, 128)
    tabp = jnp.pad(tables, ((0, 0), (0, _VP - _V - 1), (0, 96))).reshape(-1, 128)

    mesh = plsc.VectorSubcoreMesh(core_axis_name="c", subcore_axis_name="s")

    @pl.kernel(
        mesh=mesh,
        compiler_params=pltpu.CompilerParams(use_tc_tiling_on_sc=True),
        out_type=jax.ShapeDtypeStruct((B * _FEAT // 128, 128), jnp.float32),
        scratch_types=[
            pltpu.VMEM((IRW, 128), jnp.int32),
            pltpu.VMEM((CL,), jnp.int32),
            pltpu.VMEM((CL, 128), jnp.float32),
            pltpu.VMEM((ORC, 128), jnp.float32),
            pltpu.SemaphoreType.DMA,
        ],
    )
    def gather(idx_hbm, tab_hbm, out_hbm, idx_v, gsc_v, emb_v, outb_v, gsem):
        wid = lax.axis_index("s") * NC + lax.axis_index("c")
        pltpu.sync_copy(idx_hbm.at[pl.ds(wid * IRW, IRW)], idx_v)

        def chunk_body(c, carry):
            for k in range(CL // _L):
                gsc_v[pl.ds(k * _L, _L)] = idx_v[c, pl.ds(k * _L, _L)]
            pltpu.async_copy(tab_hbm.at[gsc_v], emb_v, gsem).wait()
            for lk in range(CL):
                for h in range(2):
                    outb_v[lk >> 2, pl.ds((lk & 3) * _D + h * _L, _L)] = (
                        emb_v[lk, pl.ds(h * _L, _L)])
            pltpu.sync_copy(outb_v,
                            out_hbm.at[pl.ds(wid * (LPW * _D // 128) + c * ORC,
                                             ORC)])
            return carry

        lax.fori_loop(0, NCH, chunk_body, 0)

    og = gather(idxp, tabp)
    return _layernorm(og, gamma, beta, B)
